# baseline (speedup 1.0000x reference)
import functools

import jax
import jax.numpy as jnp
from jax import lax
from jax.experimental import pallas as pl
from jax.experimental.pallas import tpu as pltpu
from jax.experimental.pallas import tpu_sc as plsc

VOCAB = 1000000
BATCH = 4096
HIST = 200
EMBED = 32
TOTAL = BATCH * HIST
NUM_CORES = 2
NUM_SUBCORES = 16
NW = NUM_CORES * NUM_SUBCORES
B_PER_W = TOTAL // NW
CHUNK = 800
N_CHUNKS = B_PER_W // CHUNK
NBUF = 4
N_GROUPS = N_CHUNKS // NBUF

VB = 8192
G1 = (VOCAB + VB - 1) // VB


@functools.partial(
    pl.pallas_call,
    grid=(G1,),
    in_specs=[pl.BlockSpec((EMBED, VB), lambda i: (0, i))],
    out_specs=pl.BlockSpec((VB, 128), lambda i: (i, 0)),
    out_shape=jax.ShapeDtypeStruct((G1 * VB, 128), jnp.float32),
)
def _transpose_table(w_ref, out_ref):
    x = w_ref[...]
    out_ref[:, 0:EMBED] = x.T


@functools.partial(
    pl.pallas_call,
    grid=(HIST,),
    in_specs=[pl.BlockSpec((1, BATCH, 128), lambda t: (t, 0, 0))],
    out_specs=pl.BlockSpec((1, EMBED, BATCH), lambda t: (t, 0, 0)),
    out_shape=jax.ShapeDtypeStruct((HIST, EMBED, BATCH), jnp.float32),
)
def _retile_out(rows_ref, out_ref):
    x = rows_ref[0]
    out_ref[...] = x[:, 0:EMBED].T[None]


_mesh = plsc.VectorSubcoreMesh(core_axis_name="c", subcore_axis_name="s")


@functools.partial(
    pl.kernel,
    mesh=_mesh,
    out_type=jax.ShapeDtypeStruct((TOTAL, 128), jnp.float32),
    scratch_types=[
        pltpu.VMEM((NBUF, CHUNK), jnp.int32),
        pltpu.VMEM((NBUF, CHUNK, EMBED), jnp.float32),
        pltpu.SemaphoreType.DMA((NBUF,)),
        pltpu.SemaphoreType.DMA((NBUF,)),
    ],
    compiler_params=pltpu.CompilerParams(use_tc_tiling_on_sc=False),
)
def _embed_gather(tokens_hbm, table_hbm, out_hbm, idx_v, rows_v, gsem, ssem):
    wid = lax.axis_index("s") * NUM_CORES + lax.axis_index("c")
    base = wid * B_PER_W

    def start_gather(i, b):
        off = base + i * CHUNK
        pltpu.sync_copy(tokens_hbm.at[pl.ds(off, CHUNK)], idx_v.at[b])
        pltpu.async_copy(table_hbm.at[idx_v.at[b]], rows_v.at[b], gsem.at[b])

    def start_store(i, b):
        off = base + i * CHUNK
        pltpu.async_copy(
            rows_v.at[b],
            out_hbm.at[pl.ds(off, CHUNK), pl.ds(0, EMBED)],
            ssem.at[b],
        )

    def wait_gather(b):
        pltpu.make_async_copy(
            table_hbm.at[idx_v.at[b]], rows_v.at[b], gsem.at[b]
        ).wait()

    def wait_store(i, b):
        off = base + i * CHUNK
        pltpu.make_async_copy(
            rows_v.at[b],
            out_hbm.at[pl.ds(off, CHUNK), pl.ds(0, EMBED)],
            ssem.at[b],
        ).wait()

    for b in range(NBUF):
        start_gather(b, b)

    def body(g, carry):
        for b in range(NBUF):
            i = g * NBUF + b
            wait_gather(b)
            start_store(i, b)
            wait_store(i, b)
            start_gather(i + NBUF, b)
        return carry

    lax.fori_loop(0, N_GROUPS - 1, body, 0)

    for b in range(NBUF):
        i = (N_GROUPS - 1) * NBUF + b
        wait_gather(b)
        start_store(i, b)
    for b in range(NBUF):
        wait_store((N_GROUPS - 1) * NBUF + b, b)


def kernel(tokens, embedding_weight):
    w_pad = _transpose_table(embedding_weight.T)
    w_rm = w_pad.reshape(G1 * VB * 4, EMBED)
    tok_flat = tokens.T.reshape(TOTAL).astype(jnp.int32) * 4
    out_pad = _embed_gather(tok_flat, w_rm)
    out_t = _retile_out(out_pad.reshape(HIST, BATCH, 128))
    return out_t.transpose(2, 0, 1)


# speedup vs baseline: 1.1779x; 1.1779x over previous
import functools

import jax
import jax.numpy as jnp
from jax import lax
from jax.experimental import pallas as pl
from jax.experimental.pallas import tpu as pltpu
from jax.experimental.pallas import tpu_sc as plsc

VOCAB = 1000000
BATCH = 4096
HIST = 200
EMBED = 32
TOTAL = BATCH * HIST
NUM_CORES = 2
NUM_SUBCORES = 16
NW = NUM_CORES * NUM_SUBCORES
B_PER_W = TOTAL // NW
CHUNK = 800
N_CHUNKS = B_PER_W // CHUNK
NBUF = 4
N_GROUPS = N_CHUNKS // NBUF

VB = 16384
G1 = (VOCAB + VB - 1) // VB


@functools.partial(
    pl.pallas_call,
    grid=(G1,),
    in_specs=[pl.BlockSpec((EMBED, VB), lambda i: (0, i))],
    out_specs=pl.BlockSpec((VB, 128), lambda i: (i, 0)),
    out_shape=jax.ShapeDtypeStruct((G1 * VB, 128), jnp.float32),
)
def _transpose_table(w_ref, out_ref):
    x = w_ref[...]
    out_ref[:, 0:EMBED] = x.T


TPB = 2


@functools.partial(
    pl.pallas_call,
    grid=(HIST // TPB,),
    in_specs=[pl.BlockSpec((TPB, BATCH, 128), lambda t: (t, 0, 0))],
    out_specs=pl.BlockSpec((TPB, EMBED, BATCH), lambda t: (t, 0, 0)),
    out_shape=jax.ShapeDtypeStruct((HIST, EMBED, BATCH), jnp.float32),
)
def _retile_out(rows_ref, out_ref):
    for k in range(TPB):
        x = rows_ref[k]
        out_ref[k] = x[:, 0:EMBED].T


_mesh = plsc.VectorSubcoreMesh(core_axis_name="c", subcore_axis_name="s")


@functools.partial(
    pl.kernel,
    mesh=_mesh,
    out_type=jax.ShapeDtypeStruct((TOTAL, 128), jnp.float32),
    scratch_types=[
        pltpu.VMEM((NBUF, CHUNK), jnp.int32),
        pltpu.VMEM((NBUF, CHUNK, EMBED), jnp.float32),
        pltpu.SemaphoreType.DMA((NBUF,)),
        pltpu.SemaphoreType.DMA((NBUF,)),
    ],
    compiler_params=pltpu.CompilerParams(use_tc_tiling_on_sc=False),
)
def _embed_gather(tokens_hbm, table_hbm, out_hbm, idx_v, rows_v, gsem, ssem):
    wid = lax.axis_index("s") * NUM_CORES + lax.axis_index("c")
    base = wid * B_PER_W

    def start_gather(i, b):
        off = base + i * CHUNK
        pltpu.sync_copy(tokens_hbm.at[pl.ds(off, CHUNK)], idx_v.at[b])
        pltpu.async_copy(table_hbm.at[idx_v.at[b]], rows_v.at[b], gsem.at[b])

    def start_store(i, b):
        off = base + i * CHUNK
        pltpu.async_copy(
            rows_v.at[b],
            out_hbm.at[pl.ds(off, CHUNK), pl.ds(0, EMBED)],
            ssem.at[b],
        )

    def wait_gather(b):
        pltpu.make_async_copy(
            table_hbm.at[idx_v.at[b]], rows_v.at[b], gsem.at[b]
        ).wait()

    def wait_store(i, b):
        off = base + i * CHUNK
        pltpu.make_async_copy(
            rows_v.at[b],
            out_hbm.at[pl.ds(off, CHUNK), pl.ds(0, EMBED)],
            ssem.at[b],
        ).wait()

    for b in range(NBUF):
        start_gather(b, b)

    def body(g, carry):
        for b in range(NBUF):
            i = g * NBUF + b
            wait_gather(b)
            start_store(i, b)
            wait_store(i, b)
            start_gather(i + NBUF, b)
        return carry

    lax.fori_loop(0, N_GROUPS - 1, body, 0)

    for b in range(NBUF):
        i = (N_GROUPS - 1) * NBUF + b
        wait_gather(b)
        start_store(i, b)
    for b in range(NBUF):
        wait_store((N_GROUPS - 1) * NBUF + b, b)


def kernel(tokens, embedding_weight):
    w_pad = _transpose_table(embedding_weight.T)
    w_rm = w_pad.reshape(G1 * VB * 4, EMBED)
    tok_flat = tokens.T.reshape(TOTAL).astype(jnp.int32) * 4
    out_pad = _embed_gather(tok_flat, w_rm)
    out_t = _retile_out(out_pad.reshape(HIST, BATCH, 128))
    return out_t.transpose(2, 0, 1)


# speedup vs baseline: 1.2618x; 1.0712x over previous
import functools

import jax
import jax.numpy as jnp
from jax import lax
from jax.experimental import pallas as pl
from jax.experimental.pallas import tpu as pltpu
from jax.experimental.pallas import tpu_sc as plsc

VOCAB = 1000000
BATCH = 4096
HIST = 200
EMBED = 32
TOTAL = BATCH * HIST
NUM_CORES = 2
NUM_SUBCORES = 16
NW = NUM_CORES * NUM_SUBCORES
B_PER_W = TOTAL // NW
CHUNK = 800
N_CHUNKS = B_PER_W // CHUNK
NBUF = 4
N_GROUPS = N_CHUNKS // NBUF

VB = 32768
G1 = (VOCAB + VB - 1) // VB


@functools.partial(
    pl.pallas_call,
    grid=(G1,),
    in_specs=[pl.BlockSpec((EMBED, VB), lambda i: (0, i))],
    out_specs=pl.BlockSpec((VB, 128), lambda i: (i, 0)),
    out_shape=jax.ShapeDtypeStruct((G1 * VB, 128), jnp.float32),
)
def _transpose_table(w_ref, out_ref):
    x = w_ref[...]
    out_ref[:, 0:EMBED] = x.T


TPB = 4


@functools.partial(
    pl.pallas_call,
    grid=(HIST // TPB,),
    in_specs=[pl.BlockSpec((TPB, BATCH, 128), lambda t: (t, 0, 0))],
    out_specs=pl.BlockSpec((TPB, EMBED, BATCH), lambda t: (t, 0, 0)),
    out_shape=jax.ShapeDtypeStruct((HIST, EMBED, BATCH), jnp.float32),
)
def _retile_out(rows_ref, out_ref):
    for k in range(TPB):
        x = rows_ref[k]
        out_ref[k] = x[:, 0:EMBED].T


_mesh = plsc.VectorSubcoreMesh(core_axis_name="c", subcore_axis_name="s")


@functools.partial(
    pl.kernel,
    mesh=_mesh,
    out_type=jax.ShapeDtypeStruct((TOTAL, 128), jnp.float32),
    scratch_types=[
        pltpu.VMEM((NBUF, CHUNK), jnp.int32),
        pltpu.VMEM((NBUF, CHUNK, EMBED), jnp.float32),
        pltpu.SemaphoreType.DMA((NBUF,)),
        pltpu.SemaphoreType.DMA((NBUF,)),
    ],
    compiler_params=pltpu.CompilerParams(use_tc_tiling_on_sc=False),
)
def _embed_gather(tokens_hbm, table_hbm, out_hbm, idx_v, rows_v, gsem, ssem):
    wid = lax.axis_index("s") * NUM_CORES + lax.axis_index("c")
    base = wid * B_PER_W

    def start_gather(i, b):
        off = base + i * CHUNK
        pltpu.sync_copy(tokens_hbm.at[pl.ds(off, CHUNK)], idx_v.at[b])
        pltpu.async_copy(table_hbm.at[idx_v.at[b]], rows_v.at[b], gsem.at[b])

    def start_store(i, b):
        off = base + i * CHUNK
        pltpu.async_copy(
            rows_v.at[b],
            out_hbm.at[pl.ds(off, CHUNK), pl.ds(0, EMBED)],
            ssem.at[b],
        )

    def wait_gather(b):
        pltpu.make_async_copy(
            table_hbm.at[idx_v.at[b]], rows_v.at[b], gsem.at[b]
        ).wait()

    def wait_store(i, b):
        off = base + i * CHUNK
        pltpu.make_async_copy(
            rows_v.at[b],
            out_hbm.at[pl.ds(off, CHUNK), pl.ds(0, EMBED)],
            ssem.at[b],
        ).wait()

    for b in range(NBUF):
        start_gather(b, b)

    def body(g, carry):
        for b in range(NBUF):
            i = g * NBUF + b
            wait_gather(b)
            start_store(i, b)
            wait_store(i, b)
            start_gather(i + NBUF, b)
        return carry

    lax.fori_loop(0, N_GROUPS - 1, body, 0)

    for b in range(NBUF):
        i = (N_GROUPS - 1) * NBUF + b
        wait_gather(b)
        start_store(i, b)
    for b in range(NBUF):
        wait_store((N_GROUPS - 1) * NBUF + b, b)


def kernel(tokens, embedding_weight):
    w_pad = _transpose_table(embedding_weight.T)
    w_rm = w_pad.reshape(G1 * VB * 4, EMBED)
    tok_flat = tokens.T.reshape(TOTAL).astype(jnp.int32) * 4
    out_pad = _embed_gather(tok_flat, w_rm)
    out_t = _retile_out(out_pad.reshape(HIST, BATCH, 128))
    return out_t.transpose(2, 0, 1)


# speedup vs baseline: 1.2684x; 1.0052x over previous
import functools

import jax
import jax.numpy as jnp
from jax import lax
from jax.experimental import pallas as pl
from jax.experimental.pallas import tpu as pltpu
from jax.experimental.pallas import tpu_sc as plsc

VOCAB = 1000000
BATCH = 4096
HIST = 200
EMBED = 32
TOTAL = BATCH * HIST
NUM_CORES = 2
NUM_SUBCORES = 16
NW = NUM_CORES * NUM_SUBCORES
B_PER_W = TOTAL // NW
CHUNK = 800
N_CHUNKS = B_PER_W // CHUNK
NBUF = 4
N_GROUPS = N_CHUNKS // NBUF

VB = 32768
G1 = (VOCAB + VB - 1) // VB


@functools.partial(
    pl.pallas_call,
    grid=(G1,),
    in_specs=[pl.BlockSpec((EMBED, VB), lambda i: (0, i))],
    out_specs=pl.BlockSpec((VB, 128), lambda i: (i, 0)),
    out_shape=jax.ShapeDtypeStruct((G1 * VB, 128), jnp.float32),
)
def _transpose_table(w_ref, out_ref):
    x = w_ref[...]
    out_ref[:, 0:EMBED] = x.T


TPB = 8


@functools.partial(
    pl.pallas_call,
    grid=(HIST // TPB,),
    in_specs=[pl.BlockSpec((TPB, BATCH, 128), lambda t: (t, 0, 0))],
    out_specs=pl.BlockSpec((TPB, EMBED, BATCH), lambda t: (t, 0, 0)),
    out_shape=jax.ShapeDtypeStruct((HIST, EMBED, BATCH), jnp.float32),
)
def _retile_out(rows_ref, out_ref):
    for k in range(TPB):
        x = rows_ref[k]
        out_ref[k] = x[:, 0:EMBED].T


_mesh = plsc.VectorSubcoreMesh(core_axis_name="c", subcore_axis_name="s")


@functools.partial(
    pl.kernel,
    mesh=_mesh,
    out_type=jax.ShapeDtypeStruct((TOTAL, 128), jnp.float32),
    scratch_types=[
        pltpu.VMEM((NBUF, CHUNK), jnp.int32),
        pltpu.VMEM((NBUF, CHUNK, EMBED), jnp.float32),
        pltpu.SemaphoreType.DMA((NBUF,)),
        pltpu.SemaphoreType.DMA((NBUF,)),
    ],
    compiler_params=pltpu.CompilerParams(use_tc_tiling_on_sc=False),
)
def _embed_gather(tokens_hbm, table_hbm, out_hbm, idx_v, rows_v, gsem, ssem):
    wid = lax.axis_index("s") * NUM_CORES + lax.axis_index("c")
    base = wid * B_PER_W

    def start_gather(i, b):
        off = base + i * CHUNK
        pltpu.sync_copy(tokens_hbm.at[pl.ds(off, CHUNK)], idx_v.at[b])
        pltpu.async_copy(table_hbm.at[idx_v.at[b]], rows_v.at[b], gsem.at[b])

    def start_store(i, b):
        off = base + i * CHUNK
        pltpu.async_copy(
            rows_v.at[b],
            out_hbm.at[pl.ds(off, CHUNK), pl.ds(0, EMBED)],
            ssem.at[b],
        )

    def wait_gather(b):
        pltpu.make_async_copy(
            table_hbm.at[idx_v.at[b]], rows_v.at[b], gsem.at[b]
        ).wait()

    def wait_store(i, b):
        off = base + i * CHUNK
        pltpu.make_async_copy(
            rows_v.at[b],
            out_hbm.at[pl.ds(off, CHUNK), pl.ds(0, EMBED)],
            ssem.at[b],
        ).wait()

    for b in range(NBUF):
        start_gather(b, b)

    def body(g, carry):
        for b in range(NBUF):
            i = g * NBUF + b
            wait_gather(b)
            start_store(i, b)
            wait_store(i, b)
            start_gather(i + NBUF, b)
        return carry

    lax.fori_loop(0, N_GROUPS - 1, body, 0)

    for b in range(NBUF):
        i = (N_GROUPS - 1) * NBUF + b
        wait_gather(b)
        start_store(i, b)
    for b in range(NBUF):
        wait_store((N_GROUPS - 1) * NBUF + b, b)


def kernel(tokens, embedding_weight):
    w_pad = _transpose_table(embedding_weight.T)
    w_rm = w_pad.reshape(G1 * VB * 4, EMBED)
    tok_flat = tokens.T.reshape(TOTAL).astype(jnp.int32) * 4
    out_pad = _embed_gather(tok_flat, w_rm)
    out_t = _retile_out(out_pad.reshape(HIST, BATCH, 128))
    return out_t.transpose(2, 0, 1)
